# baseline (speedup 1.0000x reference)
import jax
import jax.numpy as jnp
from jax import lax
from jax.experimental import pallas as pl
from jax.experimental.pallas import tpu as pltpu
from jax.experimental.pallas import tpu_sc as plsc

ORG_VOCAB = 32000
RANK = 16
EMBED_DIM = 4096
MAX_LORAS = 8

NC, NS, L = 2, 16, 16
NW = NC * NS
CH = 8
IDX_CH = 128
RP = 168


def _sc_wgather(x_h, i0_h, w_h, rows_h,
                xv, i0v, idxb, buf0, buf1,
                gs0, gs1, ss0, ss1):
    tpw = xv.shape[0]
    nch = tpw // CH
    wid = lax.axis_index("s") * NC + lax.axis_index("c")
    base = wid * tpw
    pltpu.sync_copy(x_h.at[pl.ds(base, tpw)], xv)
    pltpu.sync_copy(i0_h.at[pl.ds(base, tpw)], i0v)
    for i in range(tpw // L):
        s = pl.ds(i * L, L)
        xi = xv[s]
        added = xi > (ORG_VOCAB - 1)
        idxb[s] = xi + jnp.where(added, i0v[s], 0)

    def gather(c, buf, sem):
        return pltpu.async_copy(w_h.at[idxb.at[pl.ds(c * CH, CH)]], buf, sem)

    def gather_wait(c, buf, sem):
        pltpu.make_async_copy(w_h.at[idxb.at[pl.ds(c * CH, CH)]], buf,
                              sem).wait()

    def scatter(c, buf, sem):
        return pltpu.async_copy(buf, rows_h.at[pl.ds(base + c * CH, CH)], sem)

    gather(0, buf0, gs0)
    gather(1, buf1, gs1)

    def body(p, carry):
        c = 2 * p
        gather_wait(c, buf0, gs0)
        s0 = scatter(c, buf0, ss0)
        gather_wait(c + 1, buf1, gs1)
        s1 = scatter(c + 1, buf1, ss1)
        s0.wait()
        gather(c + 2, buf0, gs0)
        s1.wait()
        gather(c + 3, buf1, gs1)
        return carry

    lax.fori_loop(0, nch // 2 - 1, body, 0)

    c = nch - 2
    gather_wait(c, buf0, gs0)
    s0 = scatter(c, buf0, ss0)
    gather_wait(c + 1, buf1, gs1)
    s1 = scatter(c + 1, buf1, ss1)
    s0.wait()
    s1.wait()


def _sc_agather(x_h, i1_h, a128_h, faw_h, sub_h, xv, i1v, idxa, subv, fawv,
                gsem):
    tpw = xv.shape[0]
    wid = lax.axis_index("s") * NC + lax.axis_index("c")
    base = wid * tpw
    pltpu.sync_copy(x_h.at[pl.ds(base, tpw)], xv)
    pltpu.sync_copy(i1_h.at[pl.ds(base, tpw)], i1v)
    for i in range(tpw // L):
        s = pl.ds(i * L, L)
        ia = xv[s] + i1v[s]
        idxa[s] = ia >> 3
        subv[s] = ia & 7
    for h in range(tpw // IDX_CH):
        pltpu.async_copy(a128_h.at[idxa.at[pl.ds(h * IDX_CH, IDX_CH)]],
                         fawv.at[pl.ds(h * IDX_CH, IDX_CH)], gsem).wait()
    pltpu.sync_copy(fawv, faw_h.at[pl.ds(base, tpw)])
    pltpu.sync_copy(subv, sub_h.at[pl.ds(base, tpw)])


def _tc_repack(a_ref, out_ref):
    parts = [a_ref[:, j, :] for j in range(8)]
    out_ref[...] = jnp.concatenate(parts, axis=1)


def _tc_lora_add(rows_ref, faw_ref, sub_ref, g_ref, bt_ref, out_ref):
    tb = faw_ref.shape[0]
    sub = sub_ref[...]
    fa = jnp.zeros((tb, RANK), jnp.float32)
    for s in range(8):
        m = (sub == s).astype(jnp.float32)
        fa = fa + faw_ref[:, s * RANK:(s + 1) * RANK] * m
    fa_rep = jnp.concatenate([fa] * MAX_LORAS, axis=1)
    grp = lax.broadcasted_iota(jnp.int32, (tb, MAX_LORAS * RANK), 1) // RANK
    sel = (grp == g_ref[...]).astype(jnp.float32)
    a_big = fa_rep * sel
    delta = jnp.dot(a_big, bt_ref[...], preferred_element_type=jnp.float32)
    out_ref[...] = rows_ref[...] + delta


def kernel(x, embeddings_indices, base_indices, weight, lora_a_stacked_2d,
           lora_b_stacked):
    batch, seq = x.shape
    t = batch * seq
    tpw = t // NW
    d = weight.shape[1]
    nq = lora_a_stacked_2d.shape[0] // 8
    x_f = x.reshape(t)
    i0 = embeddings_indices[0].reshape(t)
    i1 = embeddings_indices[1].reshape(t)

    mesh = plsc.VectorSubcoreMesh(core_axis_name="c", subcore_axis_name="s")

    sc_w = pl.kernel(
        _sc_wgather,
        out_type=jax.ShapeDtypeStruct((t, d), jnp.float32),
        mesh=mesh,
        scratch_types=(
            pltpu.VMEM((tpw,), jnp.int32),
            pltpu.VMEM((tpw,), jnp.int32),
            pltpu.VMEM((tpw,), jnp.int32),
            pltpu.VMEM((CH, d), jnp.float32),
            pltpu.VMEM((CH, d), jnp.float32),
            pltpu.SemaphoreType.DMA,
            pltpu.SemaphoreType.DMA,
            pltpu.SemaphoreType.DMA,
            pltpu.SemaphoreType.DMA,
        ),
    )
    rows = sc_w(x_f, i0, weight)

    rb = 512
    a128 = pl.pallas_call(
        _tc_repack,
        grid=(nq // rb,),
        in_specs=[pl.BlockSpec((rb, 8, RANK), lambda i: (i, 0, 0))],
        out_specs=pl.BlockSpec((rb, 8 * RANK), lambda i: (i, 0)),
        out_shape=jax.ShapeDtypeStruct((nq, 8 * RANK), jnp.float32),
    )(lora_a_stacked_2d.reshape(nq, 8, RANK))

    sc_a = pl.kernel(
        _sc_agather,
        out_type=(
            jax.ShapeDtypeStruct((t, 8 * RANK), jnp.float32),
            jax.ShapeDtypeStruct((t,), jnp.int32),
        ),
        mesh=mesh,
        scratch_types=(
            pltpu.VMEM((tpw,), jnp.int32),
            pltpu.VMEM((tpw,), jnp.int32),
            pltpu.VMEM((tpw,), jnp.int32),
            pltpu.VMEM((tpw,), jnp.int32),
            pltpu.VMEM((tpw, 8 * RANK), jnp.float32),
            pltpu.SemaphoreType.DMA,
        ),
    )
    faw, sub = sc_a(x_f, i1, a128)

    bt = lora_b_stacked[:, 0].transpose(0, 2, 1).reshape(MAX_LORAS * RANK, d)
    g2 = base_indices.reshape(t, 1)
    sub2 = sub.reshape(t, 1)

    tb = 256
    out = pl.pallas_call(
        _tc_lora_add,
        grid=(t // tb,),
        in_specs=[
            pl.BlockSpec((tb, d), lambda i: (i, 0)),
            pl.BlockSpec((tb, 8 * RANK), lambda i: (i, 0)),
            pl.BlockSpec((tb, 1), lambda i: (i, 0)),
            pl.BlockSpec((tb, 1), lambda i: (i, 0)),
            pl.BlockSpec((MAX_LORAS * RANK, d), lambda i: (0, 0)),
        ],
        out_specs=pl.BlockSpec((tb, d), lambda i: (i, 0)),
        out_shape=jax.ShapeDtypeStruct((t, d), jnp.float32),
    )(rows, faw, sub2, g2, bt)
    return out.reshape(batch, seq, d)


# speedup vs baseline: 1.1543x; 1.1543x over previous
import jax
import jax.numpy as jnp
from jax import lax
from jax.experimental import pallas as pl
from jax.experimental.pallas import tpu as pltpu
from jax.experimental.pallas import tpu_sc as plsc

ORG_VOCAB = 32000
RANK = 16
EMBED_DIM = 4096
MAX_LORAS = 8

NC, NS, L = 2, 16, 16
NW = NC * NS
CH = 8
IDX_CH = 128
RP = 168


def _sc_wgather(x_h, i0_h, w_h, rows_h,
                xv, i0v, idxb, buf0, buf1,
                gs0, gs1, ss0, ss1):
    tpw = xv.shape[0]
    nch = tpw // CH
    wid = lax.axis_index("s") * NC + lax.axis_index("c")
    base = wid * tpw
    pltpu.sync_copy(x_h.at[pl.ds(base, tpw)], xv)
    pltpu.sync_copy(i0_h.at[pl.ds(base, tpw)], i0v)
    for i in range(tpw // L):
        s = pl.ds(i * L, L)
        xi = xv[s]
        added = xi > (ORG_VOCAB - 1)
        idxb[s] = xi + jnp.where(added, i0v[s], 0)

    def gather(c, buf, sem):
        return pltpu.async_copy(w_h.at[idxb.at[pl.ds(c * CH, CH)]], buf, sem)

    def gather_wait(c, buf, sem):
        pltpu.make_async_copy(w_h.at[idxb.at[pl.ds(c * CH, CH)]], buf,
                              sem).wait()

    def scatter(c, buf, sem):
        return pltpu.async_copy(buf, rows_h.at[pl.ds(base + c * CH, CH)], sem)

    gather(0, buf0, gs0)
    gather(1, buf1, gs1)

    def body(p, carry):
        c = 2 * p
        gather_wait(c, buf0, gs0)
        s0 = scatter(c, buf0, ss0)
        gather_wait(c + 1, buf1, gs1)
        s1 = scatter(c + 1, buf1, ss1)
        s0.wait()
        gather(c + 2, buf0, gs0)
        s1.wait()
        gather(c + 3, buf1, gs1)
        return carry

    lax.fori_loop(0, nch // 2 - 1, body, 0)

    c = nch - 2
    gather_wait(c, buf0, gs0)
    s0 = scatter(c, buf0, ss0)
    gather_wait(c + 1, buf1, gs1)
    s1 = scatter(c + 1, buf1, ss1)
    s0.wait()
    s1.wait()


def _sc_agather(x_h, i1_h, a128_h, _dep_h, faw_h, sub_h,
                xv, i1v, idxa, subv, fawv, gsem):
    tpw = xv.shape[0]
    wid = lax.axis_index("s") * NC + lax.axis_index("c")
    base = wid * tpw
    pltpu.sync_copy(x_h.at[pl.ds(base, tpw)], xv)
    pltpu.sync_copy(i1_h.at[pl.ds(base, tpw)], i1v)
    for i in range(tpw // L):
        s = pl.ds(i * L, L)
        ia = xv[s] + i1v[s]
        idxa[s] = ia >> 3
        subv[s] = ia & 7
    for h in range(tpw // IDX_CH):
        pltpu.async_copy(a128_h.at[idxa.at[pl.ds(h * IDX_CH, IDX_CH)]],
                         fawv.at[pl.ds(h * IDX_CH, IDX_CH)], gsem).wait()
    pltpu.sync_copy(fawv, faw_h.at[pl.ds(base, tpw)])
    pltpu.sync_copy(subv, sub_h.at[pl.ds(base, tpw)])


def _tc_repack(a_ref, out_ref):
    parts = [a_ref[:, j, :] for j in range(8)]
    out_ref[...] = jnp.concatenate(parts, axis=1)


def _tc_lora_add(rows_ref, faw_ref, sub_ref, g_ref, bt_ref, out_ref):
    tb = faw_ref.shape[0]
    sub = sub_ref[...]
    fa = jnp.zeros((tb, RANK), jnp.float32)
    for s in range(8):
        m = (sub == s).astype(jnp.float32)
        fa = fa + faw_ref[:, s * RANK:(s + 1) * RANK] * m
    fa_rep = jnp.concatenate([fa] * MAX_LORAS, axis=1)
    grp = lax.broadcasted_iota(jnp.int32, (tb, MAX_LORAS * RANK), 1) // RANK
    sel = (grp == g_ref[...]).astype(jnp.float32)
    a_big = fa_rep * sel
    delta = jnp.dot(a_big, bt_ref[...], preferred_element_type=jnp.float32)
    out_ref[...] = rows_ref[...] + delta


def kernel(x, embeddings_indices, base_indices, weight, lora_a_stacked_2d,
           lora_b_stacked):
    batch, seq = x.shape
    t = batch * seq
    tpw = t // NW
    d = weight.shape[1]
    nq = lora_a_stacked_2d.shape[0] // 8
    x_f = x.reshape(t)
    i0 = embeddings_indices[0].reshape(t)
    i1 = embeddings_indices[1].reshape(t)

    mesh = plsc.VectorSubcoreMesh(core_axis_name="c", subcore_axis_name="s")

    sc_w = pl.kernel(
        _sc_wgather,
        out_type=jax.ShapeDtypeStruct((t, d), jnp.float32),
        mesh=mesh,
        scratch_types=(
            pltpu.VMEM((tpw,), jnp.int32),
            pltpu.VMEM((tpw,), jnp.int32),
            pltpu.VMEM((tpw,), jnp.int32),
            pltpu.VMEM((CH, d), jnp.float32),
            pltpu.VMEM((CH, d), jnp.float32),
            pltpu.SemaphoreType.DMA,
            pltpu.SemaphoreType.DMA,
            pltpu.SemaphoreType.DMA,
            pltpu.SemaphoreType.DMA,
        ),
    )
    rows = sc_w(x_f, i0, weight)

    rb = 512
    a128 = pl.pallas_call(
        _tc_repack,
        grid=(nq // rb,),
        in_specs=[pl.BlockSpec((rb, 8, RANK), lambda i: (i, 0, 0))],
        out_specs=pl.BlockSpec((rb, 8 * RANK), lambda i: (i, 0)),
        out_shape=jax.ShapeDtypeStruct((nq, 8 * RANK), jnp.float32),
    )(lora_a_stacked_2d.reshape(nq, 8, RANK))

    sc_a = pl.kernel(
        _sc_agather,
        out_type=(
            jax.ShapeDtypeStruct((t, 8 * RANK), jnp.float32),
            jax.ShapeDtypeStruct((t,), jnp.int32),
        ),
        mesh=mesh,
        scratch_types=(
            pltpu.VMEM((tpw,), jnp.int32),
            pltpu.VMEM((tpw,), jnp.int32),
            pltpu.VMEM((tpw,), jnp.int32),
            pltpu.VMEM((tpw,), jnp.int32),
            pltpu.VMEM((tpw, 8 * RANK), jnp.float32),
            pltpu.SemaphoreType.DMA,
        ),
    )
    faw, sub = sc_a(x_f, i1, a128, lax.slice(rows, (0, 0), (8, 128)))

    bt = lora_b_stacked[:, 0].transpose(0, 2, 1).reshape(MAX_LORAS * RANK, d)
    g2 = base_indices.reshape(t, 1)
    sub2 = sub.reshape(t, 1)

    tb = 256
    out = pl.pallas_call(
        _tc_lora_add,
        grid=(t // tb,),
        in_specs=[
            pl.BlockSpec((tb, d), lambda i: (i, 0)),
            pl.BlockSpec((tb, 8 * RANK), lambda i: (i, 0)),
            pl.BlockSpec((tb, 1), lambda i: (i, 0)),
            pl.BlockSpec((tb, 1), lambda i: (i, 0)),
            pl.BlockSpec((MAX_LORAS * RANK, d), lambda i: (0, 0)),
        ],
        out_specs=pl.BlockSpec((tb, d), lambda i: (i, 0)),
        out_shape=jax.ShapeDtypeStruct((t, d), jnp.float32),
    )(rows, faw, sub2, g2, bt)
    return out.reshape(batch, seq, d)


# speedup vs baseline: 1.2216x; 1.0582x over previous
import jax
import jax.numpy as jnp
from jax import lax
from jax.experimental import pallas as pl
from jax.experimental.pallas import tpu as pltpu
from jax.experimental.pallas import tpu_sc as plsc

ORG_VOCAB = 32000
RANK = 16
EMBED_DIM = 4096
MAX_LORAS = 8

NC, NS, L = 2, 16, 16
NW = NC * NS
CH = 8
IDX_CH = 128


def _sc_wgather(x_h, i0_h, w_h, rows_h,
                xv, i0v, idxb, buf0, buf1,
                gs0, gs1, ss0, ss1):
    tpw = xv.shape[0]
    nch = tpw // CH
    wid = lax.axis_index("s") * NC + lax.axis_index("c")
    base = wid * tpw
    pltpu.sync_copy(x_h.at[pl.ds(base, tpw)], xv)
    pltpu.sync_copy(i0_h.at[pl.ds(base, tpw)], i0v)
    for i in range(tpw // L):
        s = pl.ds(i * L, L)
        xi = xv[s]
        added = xi > (ORG_VOCAB - 1)
        idxb[s] = xi + jnp.where(added, i0v[s], 0)

    def gather(c, buf, sem):
        return pltpu.async_copy(w_h.at[idxb.at[pl.ds(c * CH, CH)]], buf, sem)

    def gather_wait(c, buf, sem):
        pltpu.make_async_copy(w_h.at[idxb.at[pl.ds(c * CH, CH)]], buf,
                              sem).wait()

    def scatter(c, buf, sem):
        return pltpu.async_copy(buf, rows_h.at[pl.ds(base + c * CH, CH)], sem)

    gather(0, buf0, gs0)
    gather(1, buf1, gs1)

    def body(p, carry):
        c = 2 * p
        gather_wait(c, buf0, gs0)
        s0 = scatter(c, buf0, ss0)
        gather_wait(c + 1, buf1, gs1)
        s1 = scatter(c + 1, buf1, ss1)
        s0.wait()
        gather(c + 2, buf0, gs0)
        s1.wait()
        gather(c + 3, buf1, gs1)
        return carry

    lax.fori_loop(0, nch // 2 - 1, body, 0)

    c = nch - 2
    gather_wait(c, buf0, gs0)
    s0 = scatter(c, buf0, ss0)
    gather_wait(c + 1, buf1, gs1)
    s1 = scatter(c + 1, buf1, ss1)
    s0.wait()
    s1.wait()


def _sc_agather(x_h, i1_h, a_h, _dep_h, fa_h, xv, i1v, idxa, fav, gsem):
    tpw = xv.shape[0]
    wid = lax.axis_index("s") * NC + lax.axis_index("c")
    base = wid * tpw
    pltpu.sync_copy(x_h.at[pl.ds(base, tpw)], xv)
    pltpu.sync_copy(i1_h.at[pl.ds(base, tpw)], i1v)
    for i in range(tpw // L):
        s = pl.ds(i * L, L)
        idxa[s] = xv[s] + i1v[s]
    for h in range(tpw // IDX_CH):
        pltpu.async_copy(a_h.at[idxa.at[pl.ds(h * IDX_CH, IDX_CH)]],
                         fav.at[pl.ds(h * IDX_CH, IDX_CH)], gsem).wait()
    pltpu.sync_copy(fav, fa_h.at[pl.ds(base, tpw)])


def _tc_lora_add(rows_ref, fa_ref, g_ref, bt_ref, out_ref):
    tb = fa_ref.shape[0]
    fa = fa_ref[...]
    fa_rep = jnp.concatenate([fa] * MAX_LORAS, axis=1)
    grp = lax.broadcasted_iota(jnp.int32, (tb, MAX_LORAS * RANK), 1) // RANK
    sel = (grp == g_ref[...]).astype(jnp.float32)
    a_big = fa_rep * sel
    delta = jnp.dot(a_big, bt_ref[...], preferred_element_type=jnp.float32)
    out_ref[...] = rows_ref[...] + delta


def kernel(x, embeddings_indices, base_indices, weight, lora_a_stacked_2d,
           lora_b_stacked):
    batch, seq = x.shape
    t = batch * seq
    tpw = t // NW
    d = weight.shape[1]
    x_f = x.reshape(t)
    i0 = embeddings_indices[0].reshape(t)
    i1 = embeddings_indices[1].reshape(t)

    mesh = plsc.VectorSubcoreMesh(core_axis_name="c", subcore_axis_name="s")

    sc_w = pl.kernel(
        _sc_wgather,
        out_type=jax.ShapeDtypeStruct((t, d), jnp.float32),
        mesh=mesh,
        scratch_types=(
            pltpu.VMEM((tpw,), jnp.int32),
            pltpu.VMEM((tpw,), jnp.int32),
            pltpu.VMEM((tpw,), jnp.int32),
            pltpu.VMEM((CH, d), jnp.float32),
            pltpu.VMEM((CH, d), jnp.float32),
            pltpu.SemaphoreType.DMA,
            pltpu.SemaphoreType.DMA,
            pltpu.SemaphoreType.DMA,
            pltpu.SemaphoreType.DMA,
        ),
    )
    rows = sc_w(x_f, i0, weight)

    sc_a = pl.kernel(
        _sc_agather,
        out_type=jax.ShapeDtypeStruct((t, RANK), jnp.float32),
        mesh=mesh,
        scratch_types=(
            pltpu.VMEM((tpw,), jnp.int32),
            pltpu.VMEM((tpw,), jnp.int32),
            pltpu.VMEM((tpw,), jnp.int32),
            pltpu.VMEM((tpw, RANK), jnp.float32),
            pltpu.SemaphoreType.DMA,
        ),
        compiler_params=pltpu.CompilerParams(use_tc_tiling_on_sc=False),
    )
    fa = sc_a(x_f, i1, lora_a_stacked_2d, lax.slice(rows, (0, 0), (8, 128)))

    bt = lora_b_stacked[:, 0].transpose(0, 2, 1).reshape(MAX_LORAS * RANK, d)
    g2 = base_indices.reshape(t, 1)

    tb = 256
    out = pl.pallas_call(
        _tc_lora_add,
        grid=(t // tb,),
        in_specs=[
            pl.BlockSpec((tb, d), lambda i: (i, 0)),
            pl.BlockSpec((tb, RANK), lambda i: (i, 0)),
            pl.BlockSpec((tb, 1), lambda i: (i, 0)),
            pl.BlockSpec((MAX_LORAS * RANK, d), lambda i: (0, 0)),
        ],
        out_specs=pl.BlockSpec((tb, d), lambda i: (i, 0)),
        out_shape=jax.ShapeDtypeStruct((t, d), jnp.float32),
    )(rows, fa, g2, bt)
    return out.reshape(batch, seq, d)
